# Initial kernel scaffold; baseline (speedup 1.0000x reference)
import jax
import jax.numpy as jnp
from jax.experimental import pallas as pl


def kernel(input, embed):
    raise NotImplementedError("write your pallas kernel here")


# speedup vs baseline: 1.8227x; 1.8227x over previous
import functools

import jax
import jax.numpy as jnp
from jax import lax
from jax.experimental import pallas as pl
from jax.experimental.pallas import tpu as pltpu
from jax.experimental.pallas import tpu_sc as plsc

_DIM = 256
_NE = 8192
_K = 4
_R = 256
_C = 1024
_NEG = -3.0e38


def _topk_body(x_ref, e_ref, idx_ref, diff_ref, rv_ref, ri_ref, acc_ref):
  i = pl.program_id(0)
  j = pl.program_id(1)
  ni = pl.num_programs(0)
  nj = pl.num_programs(1)

  x = x_ref[...]
  e = e_ref[...]
  esq = jnp.sum(e * e, axis=0, keepdims=True)
  s = 2.0 * jnp.dot(x, e, preferred_element_type=jnp.float32) - esq

  col0 = (j * _C).astype(jnp.int32)
  colid = lax.broadcasted_iota(jnp.int32, (_R, _C), 1)

  tvals, tidx = [], []
  for _ in range(_K):
    m = jnp.max(s, axis=1, keepdims=True)
    a = jnp.argmax(s, axis=1).astype(jnp.int32)[:, None]
    tvals.append(m)
    tidx.append(a + col0)
    s = jnp.where(colid == a, _NEG, s)
  tv = jnp.concatenate(tvals, axis=1)
  ti = jnp.concatenate(tidx, axis=1)

  @pl.when(j == 0)
  def _():
    rv_ref[...] = tv
    ri_ref[...] = ti

  @pl.when(j > 0)
  def _():
    pool_v = jnp.concatenate([rv_ref[...], tv], axis=1)
    pool_i = jnp.concatenate([ri_ref[...], ti], axis=1)
    pid = lax.broadcasted_iota(jnp.int32, (_R, 2 * _K), 1)
    nv, nidx = [], []
    for _ in range(_K):
      m = jnp.max(pool_v, axis=1, keepdims=True)
      a = jnp.argmax(pool_v, axis=1).astype(jnp.int32)[:, None]
      sel = pid == a
      nv.append(m)
      nidx.append(jnp.sum(jnp.where(sel, pool_i, 0), axis=1, keepdims=True))
      pool_v = jnp.where(sel, _NEG, pool_v)
    rv_ref[...] = jnp.concatenate(nv, axis=1)
    ri_ref[...] = jnp.concatenate(nidx, axis=1)

  @pl.when(j == nj - 1)
  def _():
    idx_ref[...] = ri_ref[...]
    contrib = _K * jnp.sum(x * x) - jnp.sum(rv_ref[...])

    @pl.when(i == 0)
    def _():
      acc_ref[0] = 0.0

    acc_ref[0] += contrib

    @pl.when(i == ni - 1)
    def _():
      diff_ref[0, 0] = acc_ref[0] / jnp.float32(ni * _R * _K * _DIM)


def _topk_call(x, embed):
  n = x.shape[0]
  grid = (n // _R, _NE // _C)
  return pl.pallas_call(
      _topk_body,
      grid=grid,
      in_specs=[
          pl.BlockSpec((_R, _DIM), lambda i, j: (i, 0)),
          pl.BlockSpec((_DIM, _C), lambda i, j: (0, j)),
      ],
      out_specs=[
          pl.BlockSpec((_R, _K), lambda i, j: (i, 0)),
          pl.BlockSpec(memory_space=pltpu.SMEM),
      ],
      out_shape=[
          jax.ShapeDtypeStruct((n, _K), jnp.int32),
          jax.ShapeDtypeStruct((1, 1), jnp.float32),
      ],
      scratch_shapes=[
          pltpu.VMEM((_R, _K), jnp.float32),
          pltpu.VMEM((_R, _K), jnp.int32),
          pltpu.SMEM((1,), jnp.float32),
      ],
  )(x, embed)


_CHUNK = 128


def _make_gather(n_idx):
  info = plsc.get_sparse_core_info()
  nw = info.num_cores * info.num_subcores
  per_w = n_idx // nw
  n_chunks = per_w // _CHUNK
  mesh = plsc.VectorSubcoreMesh(core_axis_name="c", subcore_axis_name="s")

  @functools.partial(
      pl.kernel,
      mesh=mesh,
      out_type=jax.ShapeDtypeStruct((n_idx, _DIM), jnp.float32),
      scratch_types=[
          pltpu.VMEM((_CHUNK,), jnp.int32),
          pltpu.VMEM((_CHUNK, _DIM), jnp.float32),
          pltpu.SemaphoreType.DMA,
      ],
  )
  def gather(table_hbm, idx_hbm, out_hbm, idx_v, rows_v, sem):
    wid = lax.axis_index("s") * info.num_cores + lax.axis_index("c")
    base = wid * per_w

    def body(c, carry):
      off = base + c * _CHUNK
      pltpu.sync_copy(idx_hbm.at[pl.ds(off, _CHUNK)], idx_v)
      pltpu.async_copy(table_hbm.at[idx_v], rows_v, sem).wait()
      pltpu.sync_copy(rows_v, out_hbm.at[pl.ds(off, _CHUNK)])
      return carry

    lax.fori_loop(0, n_chunks, body, 0, unroll=False)

  return gather


def kernel(input, embed):
  b, h, w, dim = input.shape
  x = input.reshape(-1, dim)
  top_idx, diff = _topk_call(x, embed)
  table = embed.T
  flat_idx = top_idx.reshape(-1)
  q = _make_gather(flat_idx.shape[0])(table, flat_idx)
  quantize_topk = q.reshape(b, h, w, _K * dim)
  embed_ind = top_idx[:, 0]
  return quantize_topk, diff[0, 0], embed_ind
